# Initial kernel scaffold; baseline (speedup 1.0000x reference)
import jax
import jax.numpy as jnp
from jax.experimental import pallas as pl


def kernel(hidden_states, gate_w, shared_gate_up_w, shared_down_w, shared_expert_gate_w, w1, w2):
    raise NotImplementedError("write your pallas kernel here")


# speedup vs baseline: 2.5991x; 2.5991x over previous
import functools

import jax
import jax.numpy as jnp
from jax import lax
from jax.experimental import pallas as pl
from jax.experimental.pallas import tpu as pltpu
from jax.experimental.pallas import tpu_sc as plsc

NUM_E = 64
D_MODEL = 2048
D_FF = 1408
S_FF = 5632
N_TOK = 4096

BLK = 128
NBLK = (2 * N_TOK) // BLK + NUM_E
P_PAD = NBLK * BLK

TB_R = 1024
TB_S = 512
SCH = 512
NS = S_FF // SCH

GW = 16
CW = 8

_DN = (((1,), (1,)), ((), ()))


def _dot_t(a, b):
    return lax.dot_general(a, b, _DN, preferred_element_type=jnp.float32)


def _router_body(l_ref, i0_ref, i1_ref, w0_ref):
    l = l_ref[...]
    iota = lax.broadcasted_iota(jnp.int32, l.shape, 1)
    m0 = jnp.max(l, axis=1, keepdims=True)
    i0 = jnp.min(jnp.where(l == m0, iota, NUM_E), axis=1, keepdims=True)
    l2 = jnp.where(iota == i0, jnp.float32(-3.0e38), l)
    m1 = jnp.max(l2, axis=1, keepdims=True)
    i1 = jnp.min(jnp.where(l2 == m1, iota, NUM_E), axis=1, keepdims=True)
    i0_ref[...] = i0
    i1_ref[...] = i1
    w0_ref[...] = jax.nn.sigmoid(m0 - m1)


def _router(logits):
    return pl.pallas_call(
        _router_body,
        grid=(N_TOK // TB_R,),
        in_specs=[
            pl.BlockSpec((TB_R, NUM_E), lambda t: (t, 0)),
        ],
        out_specs=[
            pl.BlockSpec((TB_R, 1), lambda t: (t, 0)),
            pl.BlockSpec((TB_R, 1), lambda t: (t, 0)),
            pl.BlockSpec((TB_R, 1), lambda t: (t, 0)),
        ],
        out_shape=[
            jax.ShapeDtypeStruct((N_TOK, 1), jnp.int32),
            jax.ShapeDtypeStruct((N_TOK, 1), jnp.int32),
            jax.ShapeDtypeStruct((N_TOK, 1), jnp.float32),
        ],
    )(logits)


def _shared_body(x_ref, wgu_ref, wd_ref, sgw_ref, o_ref):
    s = pl.program_id(1)
    xb = x_ref[...].astype(jnp.bfloat16)
    g = _dot_t(xb, wgu_ref[0])
    u = _dot_t(xb, wgu_ref[1])
    h = (g * jax.nn.sigmoid(g) * u).astype(jnp.bfloat16)
    part = _dot_t(h, wd_ref[...])

    @pl.when(s == 0)
    def _():
        o_ref[...] = part

    @pl.when(s > 0)
    def _():
        o_ref[...] += part

    @pl.when(s == NS - 1)
    def _():
        sg = jax.nn.sigmoid(_dot_t(x_ref[...], sgw_ref[...]))
        o_ref[...] = o_ref[...] * sg


def _shared_expert(x, wgu, wd, sgw):
    return pl.pallas_call(
        _shared_body,
        grid=(N_TOK // TB_S, NS),
        in_specs=[
            pl.BlockSpec((TB_S, D_MODEL), lambda t, s: (t, 0)),
            pl.BlockSpec((2, SCH, D_MODEL), lambda t, s: (0, s, 0)),
            pl.BlockSpec((D_MODEL, SCH), lambda t, s: (0, s)),
            pl.BlockSpec((1, D_MODEL), lambda t, s: (0, 0)),
        ],
        out_specs=pl.BlockSpec((TB_S, D_MODEL), lambda t, s: (t, 0)),
        out_shape=jax.ShapeDtypeStruct((N_TOK, D_MODEL), jnp.float32),
    )(x, wgu, wd, sgw)


def _route_metadata(i0, i1, w0):
    n2 = 2 * N_TOK
    flat_e = jnp.concatenate([i0, i1], axis=1).reshape(-1)
    w_flat = jnp.concatenate([w0, 1.0 - w0], axis=1).reshape(-1)
    order = jnp.argsort(flat_e).astype(jnp.int32)
    sorted_e = flat_e[order]
    cnt = jnp.zeros((NUM_E,), jnp.int32).at[flat_e].add(1)
    nb = (cnt + BLK - 1) // BLK
    cum_nb = jnp.cumsum(nb)
    padded_base = (cum_nb - nb) * BLK
    start = jnp.cumsum(cnt) - cnt
    j = jnp.arange(n2, dtype=jnp.int32)
    ppos = padded_base[sorted_e] + (j - start[sorted_e])
    row_idx = jnp.zeros((P_PAD,), jnp.int32).at[ppos].set(order // 2)
    wp = jnp.zeros((P_PAD,), jnp.float32).at[ppos].set(w_flat[order])
    pos = jnp.zeros((n2,), jnp.int32).at[order].set(ppos).reshape(N_TOK, 2)
    bidx = jnp.arange(NBLK, dtype=jnp.int32)
    be = jnp.clip(jnp.searchsorted(cum_nb, bidx, side='right'),
                  0, NUM_E - 1).astype(jnp.int32)
    bv = (bidx < cum_nb[-1]).astype(jnp.int32)
    return row_idx, wp.reshape(P_PAD, 1), pos[:, 0], pos[:, 1], be, bv


_NW = 32


def _sc_gather(x, row_idx):
    per_w = P_PAD // _NW
    n_chunks = per_w // GW
    mesh = plsc.VectorSubcoreMesh(core_axis_name="core",
                                  subcore_axis_name="subcore")

    @functools.partial(
        pl.kernel,
        out_type=jax.ShapeDtypeStruct((P_PAD, D_MODEL), jnp.float32),
        mesh=mesh,
        scratch_types=[pltpu.VMEM((GW,), jnp.int32),
                       pltpu.VMEM((GW, D_MODEL), jnp.float32),
                       pltpu.SemaphoreType.DMA])
    def k(x_hbm, i_hbm, o_hbm, idx_v, rows_v, sem):
        wid = lax.axis_index("subcore") * 2 + lax.axis_index("core")

        @pl.loop(0, n_chunks)
        def _(c):
            base = wid * per_w + c * GW
            pltpu.sync_copy(i_hbm.at[pl.ds(base, GW)], idx_v)
            pltpu.async_copy(x_hbm.at[idx_v], rows_v, sem).wait()
            pltpu.sync_copy(rows_v, o_hbm.at[pl.ds(base, GW)])

    return k(x, row_idx)


def _group_body(be_ref, bv_ref, x_ref, w1_ref, w2_ref, wp_ref, o_ref):
    b = pl.program_id(0)

    @pl.when(bv_ref[b] > 0)
    def _():
        xb = x_ref[...].astype(jnp.bfloat16)
        gu = _dot_t(xb, w1_ref[0])
        g = gu[:, :D_FF]
        u = gu[:, D_FF:]
        h = (g * jax.nn.sigmoid(g) * u).astype(jnp.bfloat16)
        y = _dot_t(h, w2_ref[0])
        o_ref[...] = y * wp_ref[...]


def _grouped_mlp(be, bv, x_sorted, w1b, w2b, wp):
    grid_spec = pltpu.PrefetchScalarGridSpec(
        num_scalar_prefetch=2,
        grid=(NBLK,),
        in_specs=[
            pl.BlockSpec((BLK, D_MODEL), lambda b, be, bv: (b, 0)),
            pl.BlockSpec((1, 2 * D_FF, D_MODEL), lambda b, be, bv: (be[b], 0, 0)),
            pl.BlockSpec((1, D_MODEL, D_FF), lambda b, be, bv: (be[b], 0, 0)),
            pl.BlockSpec((BLK, 1), lambda b, be, bv: (b, 0)),
        ],
        out_specs=pl.BlockSpec((BLK, D_MODEL), lambda b, be, bv: (b, 0)),
    )
    return pl.pallas_call(
        _group_body,
        grid_spec=grid_spec,
        out_shape=jax.ShapeDtypeStruct((P_PAD, D_MODEL), jnp.float32),
    )(be, bv, x_sorted, w1b, w2b, wp)


def _sc_combine(y, pos0, pos1, shared):
    per_w = N_TOK // _NW
    n_chunks = per_w // CW
    mesh = plsc.VectorSubcoreMesh(core_axis_name="core",
                                  subcore_axis_name="subcore")

    @functools.partial(
        pl.kernel,
        out_type=jax.ShapeDtypeStruct((N_TOK, D_MODEL), jnp.float32),
        mesh=mesh,
        scratch_types=[pltpu.VMEM((CW,), jnp.int32),
                       pltpu.VMEM((CW,), jnp.int32),
                       pltpu.VMEM((CW, D_MODEL), jnp.float32),
                       pltpu.VMEM((CW, D_MODEL), jnp.float32),
                       pltpu.VMEM((CW, D_MODEL), jnp.float32),
                       pltpu.SemaphoreType.DMA,
                       pltpu.SemaphoreType.DMA])
    def k(y_hbm, p0_hbm, p1_hbm, sh_hbm, o_hbm,
          p0_v, p1_v, t0, t1, sh_v, sem0, sem1):
        wid = lax.axis_index("subcore") * 2 + lax.axis_index("core")

        @pl.loop(0, n_chunks)
        def _(c):
            base = wid * per_w + c * CW
            pltpu.sync_copy(p0_hbm.at[pl.ds(base, CW)], p0_v)
            pltpu.sync_copy(p1_hbm.at[pl.ds(base, CW)], p1_v)
            cp0 = pltpu.async_copy(y_hbm.at[p0_v], t0, sem0)
            cp1 = pltpu.async_copy(y_hbm.at[p1_v], t1, sem1)
            pltpu.sync_copy(sh_hbm.at[pl.ds(base, CW)], sh_v)
            cp0.wait()
            cp1.wait()

            @pl.loop(0, CW)
            def _(r):
                @pl.loop(0, D_MODEL, step=16)
                def _(cc):
                    slc = (pl.ds(r, 1), pl.ds(cc, 16))
                    t0.at[slc][...] = (t0.at[slc][...] + t1.at[slc][...]
                                       + sh_v.at[slc][...])

            pltpu.sync_copy(t0, o_hbm.at[pl.ds(base, CW)])

    return k(y, pos0, pos1, shared)


def kernel(hidden_states, gate_w, shared_gate_up_w, shared_down_w,
           shared_expert_gate_w, w1, w2):
    x = hidden_states
    wgu = shared_gate_up_w.reshape(2, S_FF, D_MODEL).astype(jnp.bfloat16)
    wd = shared_down_w.astype(jnp.bfloat16)
    w1b = w1.astype(jnp.bfloat16)
    w2b = w2.astype(jnp.bfloat16)

    router_logits = x @ gate_w.T
    i0, i1, w0 = _router(router_logits)
    shared = _shared_expert(x, wgu, wd, shared_expert_gate_w)
    row_idx, wp, pos0, pos1, be, bv = _route_metadata(i0, i1, w0)
    x_sorted = _sc_gather(x, row_idx)
    y = _grouped_mlp(be, bv, x_sorted, w1b, w2b, wp)
    return _sc_combine(y, pos0, pos1, shared)


# speedup vs baseline: 2.6910x; 1.0354x over previous
import functools

import jax
import jax.numpy as jnp
from jax import lax
from jax.experimental import pallas as pl
from jax.experimental.pallas import tpu as pltpu
from jax.experimental.pallas import tpu_sc as plsc

NUM_E = 64
D_MODEL = 2048
D_FF = 1408
S_FF = 5632
N_TOK = 4096

BLK = 128
NBLK = (2 * N_TOK) // BLK + NUM_E
P_PAD = NBLK * BLK

TB_R = 1024
TB_S = 512
SCH = 512
NS = S_FF // SCH

GW = 16
CW = 8

_DN = (((1,), (1,)), ((), ()))


def _dot_t(a, b):
    return lax.dot_general(a, b, _DN, preferred_element_type=jnp.float32)


def _router_body(l_ref, i0_ref, i1_ref, w0_ref):
    l = l_ref[...]
    iota = lax.broadcasted_iota(jnp.int32, l.shape, 1)
    m0 = jnp.max(l, axis=1, keepdims=True)
    i0 = jnp.min(jnp.where(l == m0, iota, NUM_E), axis=1, keepdims=True)
    l2 = jnp.where(iota == i0, jnp.float32(-3.0e38), l)
    m1 = jnp.max(l2, axis=1, keepdims=True)
    i1 = jnp.min(jnp.where(l2 == m1, iota, NUM_E), axis=1, keepdims=True)
    i0_ref[...] = i0
    i1_ref[...] = i1
    w0_ref[...] = jax.nn.sigmoid(m0 - m1)


def _router(logits):
    return pl.pallas_call(
        _router_body,
        grid=(N_TOK // TB_R,),
        in_specs=[
            pl.BlockSpec((TB_R, NUM_E), lambda t: (t, 0)),
        ],
        out_specs=[
            pl.BlockSpec((TB_R, 1), lambda t: (t, 0)),
            pl.BlockSpec((TB_R, 1), lambda t: (t, 0)),
            pl.BlockSpec((TB_R, 1), lambda t: (t, 0)),
        ],
        out_shape=[
            jax.ShapeDtypeStruct((N_TOK, 1), jnp.int32),
            jax.ShapeDtypeStruct((N_TOK, 1), jnp.int32),
            jax.ShapeDtypeStruct((N_TOK, 1), jnp.float32),
        ],
    )(logits)


def _shared_body(x_ref, wgu_ref, wd_ref, sgw_ref, o_ref):
    s = pl.program_id(1)
    xb = x_ref[...].astype(jnp.bfloat16)
    g = _dot_t(xb, wgu_ref[0])
    u = _dot_t(xb, wgu_ref[1])
    h = (g * jax.nn.sigmoid(g) * u).astype(jnp.bfloat16)
    part = _dot_t(h, wd_ref[...])

    @pl.when(s == 0)
    def _():
        o_ref[...] = part

    @pl.when(s > 0)
    def _():
        o_ref[...] += part

    @pl.when(s == NS - 1)
    def _():
        sg = jax.nn.sigmoid(_dot_t(x_ref[...], sgw_ref[...]))
        o_ref[...] = o_ref[...] * sg


def _shared_expert(x, wgu, wd, sgw):
    return pl.pallas_call(
        _shared_body,
        grid=(N_TOK // TB_S, NS),
        in_specs=[
            pl.BlockSpec((TB_S, D_MODEL), lambda t, s: (t, 0)),
            pl.BlockSpec((2, SCH, D_MODEL), lambda t, s: (0, s, 0)),
            pl.BlockSpec((D_MODEL, SCH), lambda t, s: (0, s)),
            pl.BlockSpec((1, D_MODEL), lambda t, s: (0, 0)),
        ],
        out_specs=pl.BlockSpec((TB_S, D_MODEL), lambda t, s: (t, 0)),
        out_shape=jax.ShapeDtypeStruct((N_TOK, D_MODEL), jnp.float32),
    )(x, wgu, wd, sgw)


def _route_metadata(i0, i1, w0):
    n2 = 2 * N_TOK
    flat_e = jnp.concatenate([i0, i1], axis=1).reshape(-1)
    w_flat = jnp.concatenate([w0, 1.0 - w0], axis=1).reshape(-1)
    order = jnp.argsort(flat_e).astype(jnp.int32)
    sorted_e = flat_e[order]
    cnt = jnp.zeros((NUM_E,), jnp.int32).at[flat_e].add(1)
    nb = (cnt + BLK - 1) // BLK
    cum_nb = jnp.cumsum(nb)
    padded_base = (cum_nb - nb) * BLK
    start = jnp.cumsum(cnt) - cnt
    j = jnp.arange(n2, dtype=jnp.int32)
    ppos = padded_base[sorted_e] + (j - start[sorted_e])
    row_idx = jnp.zeros((P_PAD,), jnp.int32).at[ppos].set(order // 2)
    wp = jnp.zeros((P_PAD,), jnp.float32).at[ppos].set(w_flat[order])
    pos = jnp.zeros((n2,), jnp.int32).at[order].set(ppos).reshape(N_TOK, 2)
    bidx = jnp.arange(NBLK, dtype=jnp.int32)
    be = jnp.clip(jnp.searchsorted(cum_nb, bidx, side='right'),
                  0, NUM_E - 1).astype(jnp.int32)
    bv = (bidx < cum_nb[-1]).astype(jnp.int32)
    return row_idx, wp.reshape(P_PAD, 1), pos[:, 0], pos[:, 1], be, bv


_NW = 32


def _sc_gather(xb, row_idx):
    per_w = P_PAD // _NW
    n_chunks = per_w // GW
    mesh = plsc.VectorSubcoreMesh(core_axis_name="core",
                                  subcore_axis_name="subcore")

    @functools.partial(
        pl.kernel,
        out_type=jax.ShapeDtypeStruct((P_PAD, D_MODEL), jnp.float32),
        mesh=mesh,
        scratch_types=[pltpu.VMEM((per_w,), jnp.int32),
                       pltpu.VMEM((GW, D_MODEL), jnp.float32),
                       pltpu.VMEM((GW, D_MODEL), jnp.float32),
                       pltpu.SemaphoreType.DMA, pltpu.SemaphoreType.DMA,
                       pltpu.SemaphoreType.DMA, pltpu.SemaphoreType.DMA])
    def k(x_hbm, i_hbm, o_hbm, idx_v, b0, b1, g0, g1, w0, w1):
        wid = lax.axis_index("subcore") * 2 + lax.axis_index("core")
        base = wid * per_w
        pltpu.sync_copy(i_hbm.at[pl.ds(base, per_w)], idx_v)
        bufs = (b0, b1)
        gsem = (g0, g1)
        wsem = (w0, w1)
        for b in range(2):
            pltpu.async_copy(x_hbm.at[idx_v.at[pl.ds(b * GW, GW)]],
                             bufs[b], gsem[b])

        @pl.loop(0, n_chunks, step=2)
        def _(c):
            for b in range(2):
                kx = c + b
                pltpu.make_async_copy(
                    x_hbm.at[idx_v.at[pl.ds(kx * GW, GW)]],
                    bufs[b], gsem[b]).wait()
                pltpu.async_copy(bufs[b],
                                 o_hbm.at[pl.ds(base + kx * GW, GW)], wsem[b])

                @pl.when(kx + 2 < n_chunks)
                def _():
                    pltpu.make_async_copy(
                        bufs[b], o_hbm.at[pl.ds(base + kx * GW, GW)],
                        wsem[b]).wait()
                    pltpu.async_copy(
                        x_hbm.at[idx_v.at[pl.ds((kx + 2) * GW, GW)]],
                        bufs[b], gsem[b])

        for b in range(2):
            kx = n_chunks - 2 + b
            pltpu.make_async_copy(
                bufs[b], o_hbm.at[pl.ds(base + kx * GW, GW)], wsem[b]).wait()

    return k(xb, row_idx)


def _group_body(be_ref, bv_ref, x_ref, w1_ref, w2_ref, wp_ref, o_ref):
    b = pl.program_id(0)

    @pl.when(bv_ref[b] > 0)
    def _():
        xb = x_ref[...].astype(jnp.bfloat16)
        gu = _dot_t(xb, w1_ref[0])
        g = gu[:, :D_FF]
        u = gu[:, D_FF:]
        h = (g * jax.nn.sigmoid(g) * u).astype(jnp.bfloat16)
        y = _dot_t(h, w2_ref[0])
        o_ref[...] = y * wp_ref[...]


def _grouped_mlp(be, bv, x_sorted, w1b, w2b, wp):
    grid_spec = pltpu.PrefetchScalarGridSpec(
        num_scalar_prefetch=2,
        grid=(NBLK,),
        in_specs=[
            pl.BlockSpec((BLK, D_MODEL), lambda b, be, bv: (b, 0)),
            pl.BlockSpec((1, 2 * D_FF, D_MODEL), lambda b, be, bv: (be[b], 0, 0)),
            pl.BlockSpec((1, D_MODEL, D_FF), lambda b, be, bv: (be[b], 0, 0)),
            pl.BlockSpec((BLK, 1), lambda b, be, bv: (b, 0)),
        ],
        out_specs=pl.BlockSpec((BLK, D_MODEL), lambda b, be, bv: (b, 0)),
    )
    return pl.pallas_call(
        _group_body,
        grid_spec=grid_spec,
        out_shape=jax.ShapeDtypeStruct((P_PAD, D_MODEL), jnp.float32),
    )(be, bv, x_sorted, w1b, w2b, wp)


def _sc_combine(y, pos0, pos1, shared):
    per_w = N_TOK // _NW
    n_chunks = per_w // CW
    mesh = plsc.VectorSubcoreMesh(core_axis_name="core",
                                  subcore_axis_name="subcore")

    @functools.partial(
        pl.kernel,
        out_type=jax.ShapeDtypeStruct((N_TOK, D_MODEL), jnp.float32),
        mesh=mesh,
        scratch_types=[pltpu.VMEM((per_w,), jnp.int32),
                       pltpu.VMEM((per_w,), jnp.int32)]
                      + 2 * [pltpu.VMEM((CW, D_MODEL), jnp.float32),
                             pltpu.VMEM((CW, D_MODEL), jnp.float32),
                             pltpu.VMEM((CW, D_MODEL), jnp.float32)]
                      + 8 * [pltpu.SemaphoreType.DMA])
    def k(y_hbm, p0_hbm, p1_hbm, sh_hbm, o_hbm,
          p0_v, p1_v, t0a, t1a, sha, t0b, t1b, shb,
          sa0, sa1, sa2, wa, sb0, sb1, sb2, wb):
        wid = lax.axis_index("subcore") * 2 + lax.axis_index("core")
        base = wid * per_w
        pltpu.sync_copy(p0_hbm.at[pl.ds(base, per_w)], p0_v)
        pltpu.sync_copy(p1_hbm.at[pl.ds(base, per_w)], p1_v)
        sets = ((t0a, t1a, sha, sa0, sa1, sa2, wa),
                (t0b, t1b, shb, sb0, sb1, sb2, wb))

        def issue(kx, t0, t1, sh_v, s0, s1, s2):
            pltpu.async_copy(y_hbm.at[p0_v.at[pl.ds(kx * CW, CW)]], t0, s0)
            pltpu.async_copy(y_hbm.at[p1_v.at[pl.ds(kx * CW, CW)]], t1, s1)
            pltpu.async_copy(sh_hbm.at[pl.ds(base + kx * CW, CW)], sh_v, s2)

        for b in range(2):
            issue(b, *sets[b][:6])

        @pl.loop(0, n_chunks, step=2)
        def _(c):
            for b in range(2):
                kx = c + b
                t0, t1, sh_v, s0, s1, s2, ws = sets[b]
                pltpu.make_async_copy(
                    y_hbm.at[p0_v.at[pl.ds(kx * CW, CW)]], t0, s0).wait()
                pltpu.make_async_copy(
                    y_hbm.at[p1_v.at[pl.ds(kx * CW, CW)]], t1, s1).wait()
                pltpu.make_async_copy(
                    sh_hbm.at[pl.ds(base + kx * CW, CW)], sh_v, s2).wait()

                @pl.loop(0, D_MODEL, step=16)
                def _(cc):
                    for r in range(CW):
                        slc = (pl.ds(r, 1), pl.ds(cc, 16))
                        t0.at[slc][...] = (t0.at[slc][...] + t1.at[slc][...]
                                           + sh_v.at[slc][...])

                pltpu.async_copy(t0, o_hbm.at[pl.ds(base + kx * CW, CW)], ws)

                @pl.when(kx + 2 < n_chunks)
                def _():
                    pltpu.make_async_copy(
                        t0, o_hbm.at[pl.ds(base + kx * CW, CW)], ws).wait()
                    issue(kx + 2, t0, t1, sh_v, s0, s1, s2)

        for b in range(2):
            kx = n_chunks - 2 + b
            t0, _t1, _sh, _s0, _s1, _s2, ws = sets[b]
            pltpu.make_async_copy(
                t0, o_hbm.at[pl.ds(base + kx * CW, CW)], ws).wait()

    return k(y, pos0, pos1, shared)


def kernel(hidden_states, gate_w, shared_gate_up_w, shared_down_w,
           shared_expert_gate_w, w1, w2):
    x = hidden_states
    wgu = shared_gate_up_w.reshape(2, S_FF, D_MODEL).astype(jnp.bfloat16)
    wd = shared_down_w.astype(jnp.bfloat16)
    w1b = w1.astype(jnp.bfloat16)
    w2b = w2.astype(jnp.bfloat16)

    router_logits = x @ gate_w.T
    i0, i1, w0 = _router(router_logits)
    shared = _shared_expert(x, wgu, wd, shared_expert_gate_w)
    row_idx, wp, pos0, pos1, be, bv = _route_metadata(i0, i1, w0)
    x_sorted = _sc_gather(x, row_idx)
    y = _grouped_mlp(be, bv, x_sorted, w1b, w2b, wp)
    return _sc_combine(y, pos0, pos1, shared)


# speedup vs baseline: 3.1360x; 1.1654x over previous
import functools

import jax
import jax.numpy as jnp
from jax import lax
from jax.experimental import pallas as pl
from jax.experimental.pallas import tpu as pltpu
from jax.experimental.pallas import tpu_sc as plsc

NUM_E = 64
D_MODEL = 2048
D_FF = 1408
S_FF = 5632
N_TOK = 4096

BLK = 128
NBLK = (2 * N_TOK) // BLK + NUM_E
P_PAD = NBLK * BLK

TB_R = 1024
TB_S = 512
SCH = 512
NS = S_FF // SCH

GW = 16
CW = 8

_DN = (((1,), (1,)), ((), ()))


def _dot_t(a, b):
    return lax.dot_general(a, b, _DN, preferred_element_type=jnp.float32)


def _router_body(l_ref, i0_ref, i1_ref, w0_ref):
    l = l_ref[...]
    iota = lax.broadcasted_iota(jnp.int32, l.shape, 1)
    m0 = jnp.max(l, axis=1, keepdims=True)
    i0 = jnp.min(jnp.where(l == m0, iota, NUM_E), axis=1, keepdims=True)
    l2 = jnp.where(iota == i0, jnp.float32(-3.0e38), l)
    m1 = jnp.max(l2, axis=1, keepdims=True)
    i1 = jnp.min(jnp.where(l2 == m1, iota, NUM_E), axis=1, keepdims=True)
    i0_ref[...] = i0
    i1_ref[...] = i1
    w0_ref[...] = jax.nn.sigmoid(m0 - m1)


def _router(logits):
    return pl.pallas_call(
        _router_body,
        grid=(N_TOK // TB_R,),
        in_specs=[
            pl.BlockSpec((TB_R, NUM_E), lambda t: (t, 0)),
        ],
        out_specs=[
            pl.BlockSpec((TB_R, 1), lambda t: (t, 0)),
            pl.BlockSpec((TB_R, 1), lambda t: (t, 0)),
            pl.BlockSpec((TB_R, 1), lambda t: (t, 0)),
        ],
        out_shape=[
            jax.ShapeDtypeStruct((N_TOK, 1), jnp.int32),
            jax.ShapeDtypeStruct((N_TOK, 1), jnp.int32),
            jax.ShapeDtypeStruct((N_TOK, 1), jnp.float32),
        ],
    )(logits)


def _shared_body(x_ref, wgu_ref, wd_ref, sgw_ref, o_ref):
    s = pl.program_id(1)
    xb = x_ref[...].astype(jnp.bfloat16)
    g = _dot_t(xb, wgu_ref[0])
    u = _dot_t(xb, wgu_ref[1])
    h = (g * jax.nn.sigmoid(g) * u).astype(jnp.bfloat16)
    part = _dot_t(h, wd_ref[...])

    @pl.when(s == 0)
    def _():
        o_ref[...] = part

    @pl.when(s > 0)
    def _():
        o_ref[...] += part

    @pl.when(s == NS - 1)
    def _():
        sg = jax.nn.sigmoid(_dot_t(x_ref[...], sgw_ref[...]))
        o_ref[...] = o_ref[...] * sg


def _shared_expert(x, wgu, wd, sgw):
    return pl.pallas_call(
        _shared_body,
        grid=(N_TOK // TB_S, NS),
        in_specs=[
            pl.BlockSpec((TB_S, D_MODEL), lambda t, s: (t, 0)),
            pl.BlockSpec((2, SCH, D_MODEL), lambda t, s: (0, s, 0)),
            pl.BlockSpec((D_MODEL, SCH), lambda t, s: (0, s)),
            pl.BlockSpec((1, D_MODEL), lambda t, s: (0, 0)),
        ],
        out_specs=pl.BlockSpec((TB_S, D_MODEL), lambda t, s: (t, 0)),
        out_shape=jax.ShapeDtypeStruct((N_TOK, D_MODEL), jnp.float32),
    )(x, wgu, wd, sgw)


def _route_metadata(i0, i1, w0):
    n2 = 2 * N_TOK
    flat_e = jnp.concatenate([i0, i1], axis=1).reshape(-1)
    w_flat = jnp.concatenate([w0, 1.0 - w0], axis=1).reshape(-1)
    order = jnp.argsort(flat_e).astype(jnp.int32)
    sorted_e = flat_e[order]
    cnt = jnp.zeros((NUM_E,), jnp.int32).at[flat_e].add(1)
    nb = (cnt + BLK - 1) // BLK
    cum_nb = jnp.cumsum(nb)
    padded_base = (cum_nb - nb) * BLK
    start = jnp.cumsum(cnt) - cnt
    j = jnp.arange(n2, dtype=jnp.int32)
    ppos = padded_base[sorted_e] + (j - start[sorted_e])
    row_idx = jnp.zeros((P_PAD,), jnp.int32).at[ppos].set(order // 2)
    wp = jnp.zeros((P_PAD,), jnp.float32).at[ppos].set(w_flat[order])
    pos = jnp.zeros((n2,), jnp.int32).at[order].set(ppos).reshape(N_TOK, 2)
    bidx = jnp.arange(NBLK, dtype=jnp.int32)
    be = jnp.clip(jnp.searchsorted(cum_nb, bidx, side='right'),
                  0, NUM_E - 1).astype(jnp.int32)
    bv = (bidx < cum_nb[-1]).astype(jnp.int32)
    return row_idx, wp.reshape(P_PAD, 1), pos[:, 0], pos[:, 1], be, bv


_NW = 32


def _sc_gather(xb, row_idx):
    per_w = P_PAD // _NW
    n_chunks = per_w // GW
    mesh = plsc.VectorSubcoreMesh(core_axis_name="core",
                                  subcore_axis_name="subcore")

    @functools.partial(
        pl.kernel,
        out_type=jax.ShapeDtypeStruct((P_PAD, D_MODEL), jnp.float32),
        mesh=mesh,
        scratch_types=[pltpu.VMEM((per_w,), jnp.int32),
                       pltpu.VMEM((GW, D_MODEL), jnp.float32),
                       pltpu.VMEM((GW, D_MODEL), jnp.float32),
                       pltpu.SemaphoreType.DMA, pltpu.SemaphoreType.DMA,
                       pltpu.SemaphoreType.DMA, pltpu.SemaphoreType.DMA])
    def k(x_hbm, i_hbm, o_hbm, idx_v, b0, b1, g0, g1, w0, w1):
        wid = lax.axis_index("subcore") * 2 + lax.axis_index("core")
        base = wid * per_w
        pltpu.sync_copy(i_hbm.at[pl.ds(base, per_w)], idx_v)
        bufs = (b0, b1)
        gsem = (g0, g1)
        wsem = (w0, w1)
        for b in range(2):
            pltpu.async_copy(x_hbm.at[idx_v.at[pl.ds(b * GW, GW)]],
                             bufs[b], gsem[b])

        @pl.loop(0, n_chunks, step=2)
        def _(c):
            for b in range(2):
                kx = c + b
                pltpu.make_async_copy(
                    x_hbm.at[idx_v.at[pl.ds(kx * GW, GW)]],
                    bufs[b], gsem[b]).wait()
                pltpu.async_copy(bufs[b],
                                 o_hbm.at[pl.ds(base + kx * GW, GW)], wsem[b])

                @pl.when(kx + 2 < n_chunks)
                def _():
                    pltpu.make_async_copy(
                        bufs[b], o_hbm.at[pl.ds(base + kx * GW, GW)],
                        wsem[b]).wait()
                    pltpu.async_copy(
                        x_hbm.at[idx_v.at[pl.ds((kx + 2) * GW, GW)]],
                        bufs[b], gsem[b])

        for b in range(2):
            kx = n_chunks - 2 + b
            pltpu.make_async_copy(
                bufs[b], o_hbm.at[pl.ds(base + kx * GW, GW)], wsem[b]).wait()

    return k(xb, row_idx)


def _group_body(be_ref, bv_ref, x_ref, w1_ref, w2_ref, wp_ref, o_ref,
                g_scr, u_scr, w2b_scr):
    b = pl.program_id(0)
    s = pl.program_id(1)

    @pl.when(bv_ref[b] > 0)
    def _():
        prev = be_ref[jnp.maximum(b - 1, 0)]

        @pl.when((s == 0) & ((b == 0) | (be_ref[b] != prev)))
        def _():
            w2b_scr[...] = w2_ref[0].astype(jnp.bfloat16)

        xh = x_ref[:, pl.ds((s % 2) * 1024, 1024)].astype(jnp.bfloat16)
        w1b = w1_ref[0, 0].astype(jnp.bfloat16)
        part = _dot_t(xh, w1b)

        @pl.when(s == 0)
        def _():
            g_scr[...] = part

        @pl.when(s == 1)
        def _():
            g_scr[...] += part

        @pl.when(s == 2)
        def _():
            u_scr[...] = part

        @pl.when(s == 3)
        def _():
            u = u_scr[...] + part
            g = g_scr[...]
            h = (g * jax.nn.sigmoid(g) * u).astype(jnp.bfloat16)
            y = _dot_t(h, w2b_scr[...])
            o_ref[...] = y * wp_ref[...]


def _grouped_mlp(be, bv, x_sorted, w1, w2, wp):
    w1v = w1.reshape(NUM_E, 2, D_FF, D_MODEL)
    grid_spec = pltpu.PrefetchScalarGridSpec(
        num_scalar_prefetch=2,
        grid=(NBLK, 4),
        in_specs=[
            pl.BlockSpec((BLK, D_MODEL), lambda b, s, be, bv: (b, 0)),
            pl.BlockSpec((1, 1, D_FF, 1024),
                         lambda b, s, be, bv: (be[b], s // 2, 0, s % 2)),
            pl.BlockSpec((1, D_MODEL, D_FF), lambda b, s, be, bv: (be[b], 0, 0)),
            pl.BlockSpec((BLK, 1), lambda b, s, be, bv: (b, 0)),
        ],
        out_specs=pl.BlockSpec((BLK, D_MODEL), lambda b, s, be, bv: (b, 0)),
        scratch_shapes=[pltpu.VMEM((BLK, D_FF), jnp.float32),
                        pltpu.VMEM((BLK, D_FF), jnp.float32),
                        pltpu.VMEM((D_MODEL, D_FF), jnp.bfloat16)],
    )
    return pl.pallas_call(
        _group_body,
        grid_spec=grid_spec,
        out_shape=jax.ShapeDtypeStruct((P_PAD, D_MODEL), jnp.float32),
    )(be, bv, x_sorted, w1v, w2, wp)


def _sc_combine(y, pos0, pos1, shared):
    per_w = N_TOK // _NW
    n_chunks = per_w // CW
    mesh = plsc.VectorSubcoreMesh(core_axis_name="core",
                                  subcore_axis_name="subcore")

    @functools.partial(
        pl.kernel,
        out_type=jax.ShapeDtypeStruct((N_TOK, D_MODEL), jnp.float32),
        mesh=mesh,
        scratch_types=[pltpu.VMEM((per_w,), jnp.int32),
                       pltpu.VMEM((per_w,), jnp.int32)]
                      + 2 * [pltpu.VMEM((CW, D_MODEL), jnp.float32),
                             pltpu.VMEM((CW, D_MODEL), jnp.float32),
                             pltpu.VMEM((CW, D_MODEL), jnp.float32)]
                      + 8 * [pltpu.SemaphoreType.DMA])
    def k(y_hbm, p0_hbm, p1_hbm, sh_hbm, o_hbm,
          p0_v, p1_v, t0a, t1a, sha, t0b, t1b, shb,
          sa0, sa1, sa2, wa, sb0, sb1, sb2, wb):
        wid = lax.axis_index("subcore") * 2 + lax.axis_index("core")
        base = wid * per_w
        pltpu.sync_copy(p0_hbm.at[pl.ds(base, per_w)], p0_v)
        pltpu.sync_copy(p1_hbm.at[pl.ds(base, per_w)], p1_v)
        sets = ((t0a, t1a, sha, sa0, sa1, sa2, wa),
                (t0b, t1b, shb, sb0, sb1, sb2, wb))

        def issue(kx, t0, t1, sh_v, s0, s1, s2):
            pltpu.async_copy(y_hbm.at[p0_v.at[pl.ds(kx * CW, CW)]], t0, s0)
            pltpu.async_copy(y_hbm.at[p1_v.at[pl.ds(kx * CW, CW)]], t1, s1)
            pltpu.async_copy(sh_hbm.at[pl.ds(base + kx * CW, CW)], sh_v, s2)

        for b in range(2):
            issue(b, *sets[b][:6])

        @pl.loop(0, n_chunks, step=2)
        def _(c):
            for b in range(2):
                kx = c + b
                t0, t1, sh_v, s0, s1, s2, ws = sets[b]
                pltpu.make_async_copy(
                    y_hbm.at[p0_v.at[pl.ds(kx * CW, CW)]], t0, s0).wait()
                pltpu.make_async_copy(
                    y_hbm.at[p1_v.at[pl.ds(kx * CW, CW)]], t1, s1).wait()
                pltpu.make_async_copy(
                    sh_hbm.at[pl.ds(base + kx * CW, CW)], sh_v, s2).wait()

                @pl.loop(0, D_MODEL, step=16)
                def _(cc):
                    for r in range(CW):
                        slc = (pl.ds(r, 1), pl.ds(cc, 16))
                        t0.at[slc][...] = (t0.at[slc][...] + t1.at[slc][...]
                                           + sh_v.at[slc][...])

                pltpu.async_copy(t0, o_hbm.at[pl.ds(base + kx * CW, CW)], ws)

                @pl.when(kx + 2 < n_chunks)
                def _():
                    pltpu.make_async_copy(
                        t0, o_hbm.at[pl.ds(base + kx * CW, CW)], ws).wait()
                    issue(kx + 2, t0, t1, sh_v, s0, s1, s2)

        for b in range(2):
            kx = n_chunks - 2 + b
            t0, _t1, _sh, _s0, _s1, _s2, ws = sets[b]
            pltpu.make_async_copy(
                t0, o_hbm.at[pl.ds(base + kx * CW, CW)], ws).wait()

    return k(y, pos0, pos1, shared)


def kernel(hidden_states, gate_w, shared_gate_up_w, shared_down_w,
           shared_expert_gate_w, w1, w2):
    x = hidden_states
    wgu = shared_gate_up_w.reshape(2, S_FF, D_MODEL).astype(jnp.bfloat16)
    wd = shared_down_w.astype(jnp.bfloat16)

    router_logits = x @ gate_w.T
    i0, i1, w0 = _router(router_logits)
    shared = _shared_expert(x, wgu, wd, shared_expert_gate_w)
    row_idx, wp, pos0, pos1, be, bv = _route_metadata(i0, i1, w0)
    x_sorted = _sc_gather(x, row_idx)
    y = _grouped_mlp(be, bv, x_sorted, w1, w2, wp)
    return _sc_combine(y, pos0, pos1, shared)
